# baseline (speedup 1.0000x reference)
import functools

import jax
import jax.numpy as jnp
from jax import lax
from jax.experimental import pallas as pl
from jax.experimental.pallas import tpu as pltpu
from jax.experimental.pallas import tpu_sc as plsc

N = 10000
E = 320000
D = 128
NC = 2
NS = 16
NW = NC * NS
CHUNK = 128
CHUNKS_PER_W = 80
EPW = CHUNKS_PER_W * CHUNK
E_PAD = EPW * NW
BLK = 40
N_PAD = N + 112
RPT = N_PAD // NS


def _deg_body(rows_hbm, ones_hbm, z16_hbm, deg_out, ridx_all, ones_v, deg_sh):
    c = lax.axis_index("c")
    s = lax.axis_index("s")
    w = c * NS + s
    pltpu.sync_copy(ones_hbm, ones_v)
    pltpu.sync_copy(rows_hbm.at[pl.ds(w * CHUNKS_PER_W, CHUNKS_PER_W)],
                    ridx_all)
    pltpu.sync_copy(z16_hbm.at[pl.ds(s * RPT, RPT)],
                    deg_sh.at[pl.ds(s * RPT, RPT)])
    plsc.subcore_barrier()

    def body(i, carry):
        pltpu.sync_copy(ones_v, deg_sh.at[ridx_all.at[i]], add=True)
        return carry

    lax.fori_loop(0, CHUNKS_PER_W, body, 0)
    plsc.subcore_barrier()
    pltpu.sync_copy(deg_sh.at[pl.ds(s * RPT, RPT)],
                    deg_out.at[c, pl.ds(s * RPT, RPT)])


def _seg_body(rows_hbm, cols_hbm, y_hbm, z128_hbm, s_out,
              ridx_v, cidx_v, gbuf, s_sh, sem):
    c = lax.axis_index("c")
    s = lax.axis_index("s")
    w = c * NS + s
    pltpu.sync_copy(z128_hbm.at[pl.ds(s * RPT, RPT)],
                    s_sh.at[pl.ds(s * RPT, RPT)])
    plsc.subcore_barrier()

    def body(i, carry):
        base = w * EPW + i * CHUNK
        pltpu.sync_copy(rows_hbm.at[pl.ds(base, CHUNK)], ridx_v)
        pltpu.sync_copy(cols_hbm.at[pl.ds(base, CHUNK)], cidx_v)
        pltpu.async_copy(y_hbm.at[ridx_v], gbuf, sem).wait()
        pltpu.sync_copy(gbuf, s_sh.at[cidx_v], add=True)
        return carry

    lax.fori_loop(0, CHUNKS_PER_W, body, 0)
    plsc.subcore_barrier()
    pltpu.sync_copy(s_sh.at[pl.ds(s * RPT, RPT)],
                    s_out.at[c, pl.ds(s * RPT, RPT)])


@functools.cache
def _sc_kernels():
    mesh = plsc.VectorSubcoreMesh(
        core_axis_name="c", subcore_axis_name="s",
        num_cores=NC, num_subcores=NS)
    deg_kernel = pl.kernel(
        _deg_body,
        out_type=jax.ShapeDtypeStruct((NC, N_PAD, 16), jnp.float32),
        mesh=mesh,
        compiler_params=pltpu.CompilerParams(use_tc_tiling_on_sc=False),
        scratch_types=[
            pltpu.VMEM((CHUNKS_PER_W, CHUNK), jnp.int32),
            pltpu.VMEM((CHUNK, 16), jnp.float32),
            pltpu.VMEM_SHARED((N_PAD, 16), jnp.float32),
        ],
    )
    seg_kernel = pl.kernel(
        _seg_body,
        out_type=jax.ShapeDtypeStruct((NC, N_PAD, D), jnp.float32),
        mesh=mesh,
        scratch_types=[
            pltpu.VMEM((CHUNK,), jnp.int32),
            pltpu.VMEM((CHUNK,), jnp.int32),
            pltpu.VMEM((CHUNK, D), jnp.float32),
            pltpu.VMEM_SHARED((N_PAD, D), jnp.float32),
            pltpu.SemaphoreType.DMA,
        ],
    )
    return deg_kernel, seg_kernel


def _y_body(x_ref, w_ref, deg_ref, y_ref):
    deg = deg_ref[0, :, 0:1] + deg_ref[1, :, 0:1] + 1.0
    dinv = lax.rsqrt(deg)
    xl = lax.dot_general(x_ref[...], w_ref[...], (((1,), (1,)), ((), ())),
                         preferred_element_type=jnp.float32)
    y_ref[...] = dinv * xl


BRB = 2528

_y_call = pl.pallas_call(
    _y_body,
    grid=(N_PAD // BRB,),
    in_specs=[
        pl.BlockSpec((BRB, D), lambda i: (i, 0)),
        pl.BlockSpec((D, D), lambda i: (0, 0)),
        pl.BlockSpec((NC, BRB, 16), lambda i: (0, i, 0)),
    ],
    out_specs=pl.BlockSpec((BRB, D), lambda i: (i, 0)),
    out_shape=jax.ShapeDtypeStruct((N_PAD, D), jnp.float32),
)


BR = 1000


def _agg_body(s_ref, y_ref, deg_ref, b_ref, agg_ref, stats_ref):
    i = pl.program_id(0)
    deg = deg_ref[0, :, 0:1] + deg_ref[1, :, 0:1] + 1.0
    dinv = lax.rsqrt(deg)
    agg = dinv * (s_ref[0] + s_ref[1] + y_ref[...]) + b_ref[...]
    agg_ref[...] = agg
    st = jnp.concatenate(
        [jnp.sum(agg, axis=0, keepdims=True),
         jnp.sum(agg * agg, axis=0, keepdims=True)], axis=0)

    @pl.when(i == 0)
    def _init():
        stats_ref[...] = st

    @pl.when(i > 0)
    def _acc():
        stats_ref[...] += st


_agg_call = pl.pallas_call(
    _agg_body,
    grid=(N // BR,),
    in_specs=[
        pl.BlockSpec((NC, BR, D), lambda i: (0, i, 0)),
        pl.BlockSpec((BR, D), lambda i: (i, 0)),
        pl.BlockSpec((NC, BR, 16), lambda i: (0, i, 0)),
        pl.BlockSpec((1, D), lambda i: (0, 0)),
    ],
    out_specs=[
        pl.BlockSpec((BR, D), lambda i: (i, 0)),
        pl.BlockSpec((2, D), lambda i: (0, 0)),
    ],
    out_shape=[
        jax.ShapeDtypeStruct((N, D), jnp.float32),
        jax.ShapeDtypeStruct((2, D), jnp.float32),
    ],
)


def _fin_body(agg_ref, stats_ref, g_ref, bt_ref, x_ref, o_ref):
    mean = stats_ref[0:1, :] * (1.0 / N)
    var = stats_ref[1:2, :] * (1.0 / N) - mean * mean
    inv = lax.rsqrt(var + 1e-5)
    h = (agg_ref[...] - mean) * inv * g_ref[...] + bt_ref[...]
    h = jnp.where(h >= 0, h, 0.1 * h)
    h = h + x_ref[...]
    o_ref[...] = jnp.where(h >= 0, h, 0.1 * h)


_fin_call = pl.pallas_call(
    _fin_body,
    grid=(N // BR,),
    in_specs=[
        pl.BlockSpec((BR, D), lambda i: (i, 0)),
        pl.BlockSpec((2, D), lambda i: (0, 0)),
        pl.BlockSpec((1, D), lambda i: (0, 0)),
        pl.BlockSpec((1, D), lambda i: (0, 0)),
        pl.BlockSpec((BR, D), lambda i: (i, 0)),
    ],
    out_specs=pl.BlockSpec((BR, D), lambda i: (i, 0)),
    out_shape=jax.ShapeDtypeStruct((N, D), jnp.float32),
)


def kernel(x, W, b, bn_gamma, bn_beta, edge_index):
    ei = edge_index.astype(jnp.int32)
    pad = jnp.full((E_PAD - E,), N, jnp.int32)
    rows = jnp.concatenate([ei[0], pad])
    cols = jnp.concatenate([ei[1], pad])
    rows2d = rows.reshape(E_PAD // CHUNK, CHUNK)
    ones16 = jnp.ones((CHUNK, 16), jnp.float32)
    z16 = jnp.zeros((N_PAD, 16), jnp.float32)
    z128 = jnp.zeros((N_PAD, D), jnp.float32)
    x_pad = jnp.concatenate([x, jnp.zeros((N_PAD - N, D), x.dtype)])

    deg_kernel, seg_kernel = _sc_kernels()
    deg = deg_kernel(rows2d, ones16, z16)
    y = _y_call(x_pad, W, deg)
    s = seg_kernel(rows, cols, y, z128)
    agg, stats = _agg_call(s, y, deg, b.reshape(1, D))
    return _fin_call(agg, stats, bn_gamma.reshape(1, D),
                     bn_beta.reshape(1, D), x)


# speedup vs baseline: 1.4662x; 1.4662x over previous
import functools

import jax
import jax.numpy as jnp
from jax import lax
from jax.experimental import pallas as pl
from jax.experimental.pallas import tpu as pltpu
from jax.experimental.pallas import tpu_sc as plsc

N = 10000
E = 320000
D = 128
NC = 2
NS = 16
NW = NC * NS
CHUNK = 128
CHUNKS_PER_W = 79
EPW = CHUNKS_PER_W * CHUNK
E_PAD = EPW * NW
N_PAD = N + 112
RPT = N_PAD // NS


def _deg_body(rows_hbm, ones_hbm, z16_hbm, deg_out, ridx_all, ones_v, deg_sh):
    c = lax.axis_index("c")
    s = lax.axis_index("s")
    w = c * NS + s
    pltpu.sync_copy(ones_hbm, ones_v)
    pltpu.sync_copy(rows_hbm.at[pl.ds(w * CHUNKS_PER_W, CHUNKS_PER_W)],
                    ridx_all)
    pltpu.sync_copy(z16_hbm.at[pl.ds(s * RPT, RPT)],
                    deg_sh.at[pl.ds(s * RPT, RPT)])
    plsc.subcore_barrier()

    def body(i, carry):
        pltpu.sync_copy(ones_v, deg_sh.at[ridx_all.at[i]], add=True)
        return carry

    lax.fori_loop(0, CHUNKS_PER_W, body, 0)
    plsc.subcore_barrier()
    pltpu.sync_copy(deg_sh.at[pl.ds(s * RPT, RPT)],
                    deg_out.at[c, pl.ds(s * RPT, RPT)])


def _seg_body(rows_hbm, cols_hbm, y_hbm, s_out,
              ridx_v, cidx_v, gbuf, s_sh, sem):
    c = lax.axis_index("c")
    s = lax.axis_index("s")
    w = c * NS + s

    zero = jnp.zeros((16,), jnp.float32)

    def zrow(i, carry):
        for j in range(D // 16):
            gbuf[i, pl.ds(j * 16, 16)] = zero
        return carry

    lax.fori_loop(0, CHUNK, zrow, 0)
    for off in range(0, RPT - CHUNK + 1, CHUNK):
        pltpu.sync_copy(gbuf, s_sh.at[pl.ds(s * RPT + off, CHUNK)])
    _tail = RPT % CHUNK
    if _tail:
        pltpu.sync_copy(gbuf.at[pl.ds(0, _tail)],
                        s_sh.at[pl.ds(s * RPT + RPT - _tail, _tail)])
    plsc.subcore_barrier()

    def body(i, carry):
        base = w * EPW + i * CHUNK
        pltpu.sync_copy(rows_hbm.at[pl.ds(base, CHUNK)], ridx_v)
        pltpu.sync_copy(cols_hbm.at[pl.ds(base, CHUNK)], cidx_v)
        pltpu.async_copy(y_hbm.at[ridx_v], gbuf, sem).wait()
        pltpu.sync_copy(gbuf, s_sh.at[cidx_v], add=True)
        return carry

    lax.fori_loop(0, CHUNKS_PER_W, body, 0)
    plsc.subcore_barrier()
    pltpu.sync_copy(s_sh.at[pl.ds(s * RPT, RPT)],
                    s_out.at[c, pl.ds(s * RPT, RPT)])


@functools.cache
def _sc_kernels():
    mesh = plsc.VectorSubcoreMesh(
        core_axis_name="c", subcore_axis_name="s",
        num_cores=NC, num_subcores=NS)
    deg_kernel = pl.kernel(
        _deg_body,
        out_type=jax.ShapeDtypeStruct((NC, N_PAD, 16), jnp.float32),
        mesh=mesh,
        compiler_params=pltpu.CompilerParams(use_tc_tiling_on_sc=False),
        scratch_types=[
            pltpu.VMEM((CHUNKS_PER_W, CHUNK), jnp.int32),
            pltpu.VMEM((CHUNK, 16), jnp.float32),
            pltpu.VMEM_SHARED((N_PAD, 16), jnp.float32),
        ],
    )
    seg_kernel = pl.kernel(
        _seg_body,
        out_type=jax.ShapeDtypeStruct((NC, N_PAD, D), jnp.float32),
        mesh=mesh,
        scratch_types=[
            pltpu.VMEM((CHUNK,), jnp.int32),
            pltpu.VMEM((CHUNK,), jnp.int32),
            pltpu.VMEM((CHUNK, D), jnp.float32),
            pltpu.VMEM_SHARED((N_PAD, D), jnp.float32),
            pltpu.SemaphoreType.DMA,
        ],
    )
    return deg_kernel, seg_kernel


def _y_body(x_ref, w_ref, deg_ref, y_ref):
    deg = deg_ref[0, :, 0:1] + deg_ref[1, :, 0:1] + 1.0
    dinv = lax.rsqrt(deg)
    xl = lax.dot_general(x_ref[...], w_ref[...], (((1,), (1,)), ((), ())),
                         preferred_element_type=jnp.float32)
    y_ref[...] = dinv * xl


BRB = 2528

_y_call = pl.pallas_call(
    _y_body,
    grid=(N_PAD // BRB,),
    in_specs=[
        pl.BlockSpec((BRB, D), lambda i: (i, 0)),
        pl.BlockSpec((D, D), lambda i: (0, 0)),
        pl.BlockSpec((NC, BRB, 16), lambda i: (0, i, 0)),
    ],
    out_specs=pl.BlockSpec((BRB, D), lambda i: (i, 0)),
    out_shape=jax.ShapeDtypeStruct((N_PAD, D), jnp.float32),
)


BR = 1000


def _agg_body(s_ref, y_ref, deg_ref, b_ref, agg_ref, stats_ref):
    i = pl.program_id(0)
    deg = deg_ref[0, :, 0:1] + deg_ref[1, :, 0:1] + 1.0
    dinv = lax.rsqrt(deg)
    agg = dinv * (s_ref[0] + s_ref[1] + y_ref[...]) + b_ref[...]
    agg_ref[...] = agg
    st = jnp.concatenate(
        [jnp.sum(agg, axis=0, keepdims=True),
         jnp.sum(agg * agg, axis=0, keepdims=True)], axis=0)

    @pl.when(i == 0)
    def _init():
        stats_ref[...] = st

    @pl.when(i > 0)
    def _acc():
        stats_ref[...] += st


_agg_call = pl.pallas_call(
    _agg_body,
    grid=(N // BR,),
    in_specs=[
        pl.BlockSpec((NC, BR, D), lambda i: (0, i, 0)),
        pl.BlockSpec((BR, D), lambda i: (i, 0)),
        pl.BlockSpec((NC, BR, 16), lambda i: (0, i, 0)),
        pl.BlockSpec((1, D), lambda i: (0, 0)),
    ],
    out_specs=[
        pl.BlockSpec((BR, D), lambda i: (i, 0)),
        pl.BlockSpec((2, D), lambda i: (0, 0)),
    ],
    out_shape=[
        jax.ShapeDtypeStruct((N, D), jnp.float32),
        jax.ShapeDtypeStruct((2, D), jnp.float32),
    ],
)


def _fin_body(agg_ref, stats_ref, g_ref, bt_ref, x_ref, o_ref):
    mean = stats_ref[0:1, :] * (1.0 / N)
    var = stats_ref[1:2, :] * (1.0 / N) - mean * mean
    inv = lax.rsqrt(var + 1e-5)
    h = (agg_ref[...] - mean) * inv * g_ref[...] + bt_ref[...]
    h = jnp.where(h >= 0, h, 0.1 * h)
    h = h + x_ref[...]
    o_ref[...] = jnp.where(h >= 0, h, 0.1 * h)


_fin_call = pl.pallas_call(
    _fin_body,
    grid=(N // BR,),
    in_specs=[
        pl.BlockSpec((BR, D), lambda i: (i, 0)),
        pl.BlockSpec((2, D), lambda i: (0, 0)),
        pl.BlockSpec((1, D), lambda i: (0, 0)),
        pl.BlockSpec((1, D), lambda i: (0, 0)),
        pl.BlockSpec((BR, D), lambda i: (i, 0)),
    ],
    out_specs=pl.BlockSpec((BR, D), lambda i: (i, 0)),
    out_shape=jax.ShapeDtypeStruct((N, D), jnp.float32),
)


def kernel(x, W, b, bn_gamma, bn_beta, edge_index):
    ei = edge_index.astype(jnp.int32)
    pad = jnp.full((E_PAD - E,), N, jnp.int32)
    rows = jnp.concatenate([ei[0], pad])
    cols = jnp.concatenate([ei[1], pad])
    rows2d = rows.reshape(E_PAD // CHUNK, CHUNK)
    ones16 = jnp.ones((CHUNK, 16), jnp.float32)
    z16 = jnp.zeros((N_PAD, 16), jnp.float32)
    x_pad = jnp.concatenate([x, jnp.zeros((N_PAD - N, D), x.dtype)])

    deg_kernel, seg_kernel = _sc_kernels()
    deg = deg_kernel(rows2d, ones16, z16)
    y = _y_call(x_pad, W, deg)
    s = seg_kernel(rows, cols, y)
    agg, stats = _agg_call(s, y, deg, b.reshape(1, D))
    return _fin_call(agg, stats, bn_gamma.reshape(1, D),
                     bn_beta.reshape(1, D), x)


# speedup vs baseline: 1.6559x; 1.1294x over previous
import functools

import jax
import jax.numpy as jnp
from jax import lax
from jax.experimental import pallas as pl
from jax.experimental.pallas import tpu as pltpu
from jax.experimental.pallas import tpu_sc as plsc

N = 10000
E = 320000
D = 128
NC = 2
NS = 16
NW = NC * NS
CHUNK = 128
CHUNKS_PER_W = 79
EPW = CHUNKS_PER_W * CHUNK
E_PAD = EPW * NW
CH0 = 101
CH1 = 2 * CHUNKS_PER_W - CH0
N_PAD = N + 112
RPT = N_PAD // NS


def _deg_body(rows_hbm, ones_hbm, z16_hbm, deg_out, ridx_all, ones_v, deg_sh):
    c = lax.axis_index("c")
    s = lax.axis_index("s")
    w = c * NS + s
    pltpu.sync_copy(ones_hbm, ones_v)
    pltpu.sync_copy(rows_hbm.at[pl.ds(w * CHUNKS_PER_W, CHUNKS_PER_W)],
                    ridx_all)
    pltpu.sync_copy(z16_hbm.at[pl.ds(s * RPT, RPT)],
                    deg_sh.at[pl.ds(s * RPT, RPT)])
    plsc.subcore_barrier()

    def body(i, carry):
        pltpu.sync_copy(ones_v, deg_sh.at[ridx_all.at[i]], add=True)
        return carry

    lax.fori_loop(0, CHUNKS_PER_W, body, 0)
    plsc.subcore_barrier()
    pltpu.sync_copy(deg_sh.at[pl.ds(s * RPT, RPT)],
                    deg_out.at[c, pl.ds(s * RPT, RPT)])


def _seg_body(rows_hbm, cols_hbm, y_hbm, s_out,
              ridx_v, cidx_v, gbuf, s_sh, sem):
    c = lax.axis_index("c")
    s = lax.axis_index("s")
    w = c * NS + s

    zero = jnp.zeros((16,), jnp.float32)

    def zrow(i, carry):
        for j in range(D // 16):
            gbuf[i, pl.ds(j * 16, 16)] = zero
        return carry

    lax.fori_loop(0, CHUNK, zrow, 0)
    for off in range(0, RPT - CHUNK + 1, CHUNK):
        pltpu.sync_copy(gbuf, s_sh.at[pl.ds(s * RPT + off, CHUNK)])
    _tail = RPT % CHUNK
    if _tail:
        pltpu.sync_copy(gbuf.at[pl.ds(0, _tail)],
                        s_sh.at[pl.ds(s * RPT + RPT - _tail, _tail)])
    plsc.subcore_barrier()

    chunk0 = jnp.where(c == 0, s * CH0, NS * CH0 + s * CH1)
    nchunks = jnp.where(c == 0, CH0, CH1)

    def body(i, carry):
        base = (chunk0 + i) * CHUNK
        pltpu.sync_copy(rows_hbm.at[pl.ds(base, CHUNK)], ridx_v)
        pltpu.sync_copy(cols_hbm.at[pl.ds(base, CHUNK)], cidx_v)
        pltpu.async_copy(y_hbm.at[ridx_v], gbuf, sem).wait()
        pltpu.sync_copy(gbuf, s_sh.at[cidx_v], add=True)
        return carry

    lax.fori_loop(0, nchunks, body, 0)
    plsc.subcore_barrier()
    pltpu.sync_copy(s_sh.at[pl.ds(s * RPT, RPT)],
                    s_out.at[c, pl.ds(s * RPT, RPT)])


@functools.cache
def _sc_kernels():
    mesh = plsc.VectorSubcoreMesh(
        core_axis_name="c", subcore_axis_name="s",
        num_cores=NC, num_subcores=NS)
    deg_kernel = pl.kernel(
        _deg_body,
        out_type=jax.ShapeDtypeStruct((NC, N_PAD, 16), jnp.float32),
        mesh=mesh,
        compiler_params=pltpu.CompilerParams(use_tc_tiling_on_sc=False),
        scratch_types=[
            pltpu.VMEM((CHUNKS_PER_W, CHUNK), jnp.int32),
            pltpu.VMEM((CHUNK, 16), jnp.float32),
            pltpu.VMEM_SHARED((N_PAD, 16), jnp.float32),
        ],
    )
    seg_kernel = pl.kernel(
        _seg_body,
        out_type=jax.ShapeDtypeStruct((NC, N_PAD, D), jnp.float32),
        mesh=mesh,
        scratch_types=[
            pltpu.VMEM((CHUNK,), jnp.int32),
            pltpu.VMEM((CHUNK,), jnp.int32),
            pltpu.VMEM((CHUNK, D), jnp.float32),
            pltpu.VMEM_SHARED((N_PAD, D), jnp.float32),
            pltpu.SemaphoreType.DMA,
        ],
    )
    return deg_kernel, seg_kernel


def _y_body(x_ref, w_ref, deg_ref, y_ref):
    deg = deg_ref[0, :, 0:1] + deg_ref[1, :, 0:1] + 1.0
    dinv = lax.rsqrt(deg)
    xl = lax.dot_general(x_ref[...], w_ref[...], (((1,), (1,)), ((), ())),
                         preferred_element_type=jnp.float32)
    y_ref[...] = dinv * xl


BRB = 2528

_y_call = pl.pallas_call(
    _y_body,
    grid=(N_PAD // BRB,),
    in_specs=[
        pl.BlockSpec((BRB, D), lambda i: (i, 0)),
        pl.BlockSpec((D, D), lambda i: (0, 0)),
        pl.BlockSpec((NC, BRB, 16), lambda i: (0, i, 0)),
    ],
    out_specs=pl.BlockSpec((BRB, D), lambda i: (i, 0)),
    out_shape=jax.ShapeDtypeStruct((N_PAD, D), jnp.float32),
)


BR = 1000


def _agg_body(s_ref, y_ref, deg_ref, b_ref, agg_ref, stats_ref):
    i = pl.program_id(0)
    deg = deg_ref[0, :, 0:1] + deg_ref[1, :, 0:1] + 1.0
    dinv = lax.rsqrt(deg)
    agg = dinv * (s_ref[0] + s_ref[1] + y_ref[...]) + b_ref[...]
    agg_ref[...] = agg
    st = jnp.concatenate(
        [jnp.sum(agg, axis=0, keepdims=True),
         jnp.sum(agg * agg, axis=0, keepdims=True)], axis=0)

    @pl.when(i == 0)
    def _init():
        stats_ref[...] = st

    @pl.when(i > 0)
    def _acc():
        stats_ref[...] += st


_agg_call = pl.pallas_call(
    _agg_body,
    grid=(N // BR,),
    in_specs=[
        pl.BlockSpec((NC, BR, D), lambda i: (0, i, 0)),
        pl.BlockSpec((BR, D), lambda i: (i, 0)),
        pl.BlockSpec((NC, BR, 16), lambda i: (0, i, 0)),
        pl.BlockSpec((1, D), lambda i: (0, 0)),
    ],
    out_specs=[
        pl.BlockSpec((BR, D), lambda i: (i, 0)),
        pl.BlockSpec((2, D), lambda i: (0, 0)),
    ],
    out_shape=[
        jax.ShapeDtypeStruct((N, D), jnp.float32),
        jax.ShapeDtypeStruct((2, D), jnp.float32),
    ],
)


def _fin_body(agg_ref, stats_ref, g_ref, bt_ref, x_ref, o_ref):
    mean = stats_ref[0:1, :] * (1.0 / N)
    var = stats_ref[1:2, :] * (1.0 / N) - mean * mean
    inv = lax.rsqrt(var + 1e-5)
    h = (agg_ref[...] - mean) * inv * g_ref[...] + bt_ref[...]
    h = jnp.where(h >= 0, h, 0.1 * h)
    h = h + x_ref[...]
    o_ref[...] = jnp.where(h >= 0, h, 0.1 * h)


_fin_call = pl.pallas_call(
    _fin_body,
    grid=(N // BR,),
    in_specs=[
        pl.BlockSpec((BR, D), lambda i: (i, 0)),
        pl.BlockSpec((2, D), lambda i: (0, 0)),
        pl.BlockSpec((1, D), lambda i: (0, 0)),
        pl.BlockSpec((1, D), lambda i: (0, 0)),
        pl.BlockSpec((BR, D), lambda i: (i, 0)),
    ],
    out_specs=pl.BlockSpec((BR, D), lambda i: (i, 0)),
    out_shape=jax.ShapeDtypeStruct((N, D), jnp.float32),
)


def kernel(x, W, b, bn_gamma, bn_beta, edge_index):
    ei = edge_index.astype(jnp.int32)
    pad = jnp.full((E_PAD - E,), N, jnp.int32)
    rows = jnp.concatenate([ei[0], pad])
    cols = jnp.concatenate([ei[1], pad])
    rows2d = rows.reshape(E_PAD // CHUNK, CHUNK)
    ones16 = jnp.ones((CHUNK, 16), jnp.float32)
    z16 = jnp.zeros((N_PAD, 16), jnp.float32)
    x_pad = jnp.concatenate([x, jnp.zeros((N_PAD - N, D), x.dtype)])

    deg_kernel, seg_kernel = _sc_kernels()
    deg = deg_kernel(rows2d, ones16, z16)
    y = _y_call(x_pad, W, deg)
    s = seg_kernel(rows, cols, y)
    agg, stats = _agg_call(s, y, deg, b.reshape(1, D))
    return _fin_call(agg, stats, bn_gamma.reshape(1, D),
                     bn_beta.reshape(1, D), x)


# speedup vs baseline: 1.7197x; 1.0385x over previous
import functools

import jax
import jax.numpy as jnp
from jax import lax
from jax.experimental import pallas as pl
from jax.experimental.pallas import tpu as pltpu
from jax.experimental.pallas import tpu_sc as plsc

N = 10000
E = 320000
D = 128
NC = 2
NS = 16
NW = NC * NS
CHUNK = 128
CHUNKS_PER_W = 79
EPW = CHUNKS_PER_W * CHUNK
E_PAD = EPW * NW
CH0 = 107
CH1 = 2 * CHUNKS_PER_W - CH0
N_PAD = N + 112
RPT = N_PAD // NS


def _deg_body(rows_hbm, ones_hbm, z16_hbm, deg_out, ridx_all, ones_v, deg_sh):
    c = lax.axis_index("c")
    s = lax.axis_index("s")
    w = c * NS + s
    pltpu.sync_copy(ones_hbm, ones_v)
    pltpu.sync_copy(rows_hbm.at[pl.ds(w * CHUNKS_PER_W, CHUNKS_PER_W)],
                    ridx_all)
    pltpu.sync_copy(z16_hbm.at[pl.ds(s * RPT, RPT)],
                    deg_sh.at[pl.ds(s * RPT, RPT)])
    plsc.subcore_barrier()

    def body(i, carry):
        pltpu.sync_copy(ones_v, deg_sh.at[ridx_all.at[i]], add=True)
        return carry

    lax.fori_loop(0, CHUNKS_PER_W, body, 0)
    plsc.subcore_barrier()
    pltpu.sync_copy(deg_sh.at[pl.ds(s * RPT, RPT)],
                    deg_out.at[c, pl.ds(s * RPT, RPT)])


def _seg_body(rows_hbm, cols_hbm, y_hbm, s_out,
              ridx_v, cidx_v, gbuf, s_sh, sem):
    c = lax.axis_index("c")
    s = lax.axis_index("s")
    w = c * NS + s

    zero = jnp.zeros((16,), jnp.float32)

    def zrow(i, carry):
        for j in range(D // 16):
            gbuf[i, pl.ds(j * 16, 16)] = zero
        return carry

    lax.fori_loop(0, CHUNK, zrow, 0)
    for off in range(0, RPT - CHUNK + 1, CHUNK):
        pltpu.sync_copy(gbuf, s_sh.at[pl.ds(s * RPT + off, CHUNK)])
    _tail = RPT % CHUNK
    if _tail:
        pltpu.sync_copy(gbuf.at[pl.ds(0, _tail)],
                        s_sh.at[pl.ds(s * RPT + RPT - _tail, _tail)])
    plsc.subcore_barrier()

    chunk0 = jnp.where(c == 0, s * CH0, NS * CH0 + s * CH1)
    nchunks = jnp.where(c == 0, CH0, CH1)

    def body(i, carry):
        base = (chunk0 + i) * CHUNK
        pltpu.sync_copy(rows_hbm.at[pl.ds(base, CHUNK)], ridx_v)
        pltpu.sync_copy(cols_hbm.at[pl.ds(base, CHUNK)], cidx_v)
        pltpu.async_copy(y_hbm.at[ridx_v], gbuf, sem).wait()
        pltpu.sync_copy(gbuf, s_sh.at[cidx_v], add=True)
        return carry

    lax.fori_loop(0, nchunks, body, 0)
    plsc.subcore_barrier()
    pltpu.sync_copy(s_sh.at[pl.ds(s * RPT, RPT)],
                    s_out.at[c, pl.ds(s * RPT, RPT)])


@functools.cache
def _sc_kernels():
    mesh = plsc.VectorSubcoreMesh(
        core_axis_name="c", subcore_axis_name="s",
        num_cores=NC, num_subcores=NS)
    deg_kernel = pl.kernel(
        _deg_body,
        out_type=jax.ShapeDtypeStruct((NC, N_PAD, 16), jnp.float32),
        mesh=mesh,
        compiler_params=pltpu.CompilerParams(use_tc_tiling_on_sc=False),
        scratch_types=[
            pltpu.VMEM((CHUNKS_PER_W, CHUNK), jnp.int32),
            pltpu.VMEM((CHUNK, 16), jnp.float32),
            pltpu.VMEM_SHARED((N_PAD, 16), jnp.float32),
        ],
    )
    seg_kernel = pl.kernel(
        _seg_body,
        out_type=jax.ShapeDtypeStruct((NC, N_PAD, D), jnp.float32),
        mesh=mesh,
        scratch_types=[
            pltpu.VMEM((CHUNK,), jnp.int32),
            pltpu.VMEM((CHUNK,), jnp.int32),
            pltpu.VMEM((CHUNK, D), jnp.float32),
            pltpu.VMEM_SHARED((N_PAD, D), jnp.float32),
            pltpu.SemaphoreType.DMA,
        ],
    )
    return deg_kernel, seg_kernel


def _y_body(x_ref, w_ref, deg_ref, y_ref):
    deg = deg_ref[0, :, 0:1] + deg_ref[1, :, 0:1] + 1.0
    dinv = lax.rsqrt(deg)
    xl = lax.dot_general(x_ref[...], w_ref[...], (((1,), (1,)), ((), ())),
                         preferred_element_type=jnp.float32)
    y_ref[...] = dinv * xl


BRB = 2528

_y_call = pl.pallas_call(
    _y_body,
    grid=(N_PAD // BRB,),
    in_specs=[
        pl.BlockSpec((BRB, D), lambda i: (i, 0)),
        pl.BlockSpec((D, D), lambda i: (0, 0)),
        pl.BlockSpec((NC, BRB, 16), lambda i: (0, i, 0)),
    ],
    out_specs=pl.BlockSpec((BRB, D), lambda i: (i, 0)),
    out_shape=jax.ShapeDtypeStruct((N_PAD, D), jnp.float32),
)


BR = 1000


def _agg_body(s_ref, y_ref, deg_ref, b_ref, agg_ref, stats_ref):
    i = pl.program_id(0)
    deg = deg_ref[0, :, 0:1] + deg_ref[1, :, 0:1] + 1.0
    dinv = lax.rsqrt(deg)
    agg = dinv * (s_ref[0] + s_ref[1] + y_ref[...]) + b_ref[...]
    agg_ref[...] = agg
    st = jnp.concatenate(
        [jnp.sum(agg, axis=0, keepdims=True),
         jnp.sum(agg * agg, axis=0, keepdims=True)], axis=0)

    @pl.when(i == 0)
    def _init():
        stats_ref[...] = st

    @pl.when(i > 0)
    def _acc():
        stats_ref[...] += st


_agg_call = pl.pallas_call(
    _agg_body,
    grid=(N // BR,),
    in_specs=[
        pl.BlockSpec((NC, BR, D), lambda i: (0, i, 0)),
        pl.BlockSpec((BR, D), lambda i: (i, 0)),
        pl.BlockSpec((NC, BR, 16), lambda i: (0, i, 0)),
        pl.BlockSpec((1, D), lambda i: (0, 0)),
    ],
    out_specs=[
        pl.BlockSpec((BR, D), lambda i: (i, 0)),
        pl.BlockSpec((2, D), lambda i: (0, 0)),
    ],
    out_shape=[
        jax.ShapeDtypeStruct((N, D), jnp.float32),
        jax.ShapeDtypeStruct((2, D), jnp.float32),
    ],
)


def _fin_body(agg_ref, stats_ref, g_ref, bt_ref, x_ref, o_ref):
    mean = stats_ref[0:1, :] * (1.0 / N)
    var = stats_ref[1:2, :] * (1.0 / N) - mean * mean
    inv = lax.rsqrt(var + 1e-5)
    h = (agg_ref[...] - mean) * inv * g_ref[...] + bt_ref[...]
    h = jnp.where(h >= 0, h, 0.1 * h)
    h = h + x_ref[...]
    o_ref[...] = jnp.where(h >= 0, h, 0.1 * h)


_fin_call = pl.pallas_call(
    _fin_body,
    grid=(N // BR,),
    in_specs=[
        pl.BlockSpec((BR, D), lambda i: (i, 0)),
        pl.BlockSpec((2, D), lambda i: (0, 0)),
        pl.BlockSpec((1, D), lambda i: (0, 0)),
        pl.BlockSpec((1, D), lambda i: (0, 0)),
        pl.BlockSpec((BR, D), lambda i: (i, 0)),
    ],
    out_specs=pl.BlockSpec((BR, D), lambda i: (i, 0)),
    out_shape=jax.ShapeDtypeStruct((N, D), jnp.float32),
)


def kernel(x, W, b, bn_gamma, bn_beta, edge_index):
    ei = edge_index.astype(jnp.int32)
    pad = jnp.full((E_PAD - E,), N, jnp.int32)
    rows = jnp.concatenate([ei[0], pad])
    cols = jnp.concatenate([ei[1], pad])
    rows2d = rows.reshape(E_PAD // CHUNK, CHUNK)
    ones16 = jnp.ones((CHUNK, 16), jnp.float32)
    z16 = jnp.zeros((N_PAD, 16), jnp.float32)
    x_pad = jnp.concatenate([x, jnp.zeros((N_PAD - N, D), x.dtype)])

    deg_kernel, seg_kernel = _sc_kernels()
    deg = deg_kernel(rows2d, ones16, z16)
    y = _y_call(x_pad, W, deg)
    s = seg_kernel(rows, cols, y)
    agg, stats = _agg_call(s, y, deg, b.reshape(1, D))
    return _fin_call(agg, stats, bn_gamma.reshape(1, D),
                     bn_beta.reshape(1, D), x)


# speedup vs baseline: 2.0512x; 1.1928x over previous
import functools

import jax
import jax.numpy as jnp
from jax import lax
from jax.experimental import pallas as pl
from jax.experimental.pallas import tpu as pltpu
from jax.experimental.pallas import tpu_sc as plsc

N = 10000
E = 320000
D = 128
NC = 2
NS = 16
NW = NC * NS
CHUNK = 128
CHUNKS_PER_W = 79
EPW = CHUNKS_PER_W * CHUNK
E_PAD = EPW * NW
CH0 = 106
CH1 = 2 * CHUNKS_PER_W - CH0
N_PAD = N + 112
RPT = N_PAD // NS


def _deg_body(rows_hbm, ones_hbm, z16_hbm, deg_out, ridx_all, ones_v, deg_sh):
    c = lax.axis_index("c")
    s = lax.axis_index("s")
    w = c * NS + s
    pltpu.sync_copy(ones_hbm, ones_v)
    pltpu.sync_copy(rows_hbm.at[pl.ds(w * CHUNKS_PER_W, CHUNKS_PER_W)],
                    ridx_all)
    pltpu.sync_copy(z16_hbm.at[pl.ds(s * RPT, RPT)],
                    deg_sh.at[pl.ds(s * RPT, RPT)])
    plsc.subcore_barrier()

    def body(i, carry):
        pltpu.sync_copy(ones_v, deg_sh.at[ridx_all.at[i]], add=True)
        return carry

    lax.fori_loop(0, CHUNKS_PER_W, body, 0)
    plsc.subcore_barrier()
    pltpu.sync_copy(deg_sh.at[pl.ds(s * RPT, RPT)],
                    deg_out.at[c, pl.ds(s * RPT, RPT)])


def _seg_body(rows_hbm, cols_hbm, y_hbm, s_out,
              ridx_v, cidx_v, ridx_v1, cidx_v1, gbuf, gbuf1, s_sh, sem, sem1):
    c = lax.axis_index("c")
    s = lax.axis_index("s")
    w = c * NS + s

    zero = jnp.zeros((16,), jnp.float32)

    def zrow(i, carry):
        for j in range(D // 16):
            gbuf[i, pl.ds(j * 16, 16)] = zero
        return carry

    lax.fori_loop(0, CHUNK, zrow, 0)
    for off in range(0, RPT - CHUNK + 1, CHUNK):
        pltpu.sync_copy(gbuf, s_sh.at[pl.ds(s * RPT + off, CHUNK)])
    _tail = RPT % CHUNK
    if _tail:
        pltpu.sync_copy(gbuf.at[pl.ds(0, _tail)],
                        s_sh.at[pl.ds(s * RPT + RPT - _tail, _tail)])
    plsc.subcore_barrier()

    chunk0 = jnp.where(c == 0, s * CH0, NS * CH0 + s * CH1)
    npairs = jnp.where(c == 0, CH0 // 2, CH1 // 2)

    def load_idx(i, ridx, cidx):
        base = (chunk0 + i) * CHUNK
        pltpu.sync_copy(rows_hbm.at[pl.ds(base, CHUNK)], ridx)
        pltpu.sync_copy(cols_hbm.at[pl.ds(base, CHUNK)], cidx)

    load_idx(0, ridx_v, cidx_v)
    pltpu.async_copy(y_hbm.at[ridx_v], gbuf, sem)

    def body(p, carry):
        i0 = 2 * p
        load_idx(i0 + 1, ridx_v1, cidx_v1)
        pltpu.async_copy(y_hbm.at[ridx_v1], gbuf1, sem1)
        pltpu.make_async_copy(y_hbm.at[ridx_v], gbuf, sem).wait()
        pltpu.sync_copy(gbuf, s_sh.at[cidx_v], add=True)

        @pl.when(p < npairs - 1)
        def _():
            load_idx(i0 + 2, ridx_v, cidx_v)
            pltpu.async_copy(y_hbm.at[ridx_v], gbuf, sem)

        pltpu.make_async_copy(y_hbm.at[ridx_v1], gbuf1, sem1).wait()
        pltpu.sync_copy(gbuf1, s_sh.at[cidx_v1], add=True)
        return carry

    lax.fori_loop(0, npairs, body, 0)
    plsc.subcore_barrier()
    pltpu.sync_copy(s_sh.at[pl.ds(s * RPT, RPT)],
                    s_out.at[c, pl.ds(s * RPT, RPT)])


@functools.cache
def _sc_kernels():
    mesh = plsc.VectorSubcoreMesh(
        core_axis_name="c", subcore_axis_name="s",
        num_cores=NC, num_subcores=NS)
    deg_kernel = pl.kernel(
        _deg_body,
        out_type=jax.ShapeDtypeStruct((NC, N_PAD, 16), jnp.float32),
        mesh=mesh,
        compiler_params=pltpu.CompilerParams(use_tc_tiling_on_sc=False),
        scratch_types=[
            pltpu.VMEM((CHUNKS_PER_W, CHUNK), jnp.int32),
            pltpu.VMEM((CHUNK, 16), jnp.float32),
            pltpu.VMEM_SHARED((N_PAD, 16), jnp.float32),
        ],
    )
    seg_kernel = pl.kernel(
        _seg_body,
        out_type=jax.ShapeDtypeStruct((NC, N_PAD, D), jnp.float32),
        mesh=mesh,
        scratch_types=[
            pltpu.VMEM((CHUNK,), jnp.int32),
            pltpu.VMEM((CHUNK,), jnp.int32),
            pltpu.VMEM((CHUNK,), jnp.int32),
            pltpu.VMEM((CHUNK,), jnp.int32),
            pltpu.VMEM((CHUNK, D), jnp.float32),
            pltpu.VMEM((CHUNK, D), jnp.float32),
            pltpu.VMEM_SHARED((N_PAD, D), jnp.float32),
            pltpu.SemaphoreType.DMA,
            pltpu.SemaphoreType.DMA,
        ],
    )
    return deg_kernel, seg_kernel


def _y_body(x_ref, w_ref, deg_ref, y_ref):
    deg = deg_ref[0, :, 0:1] + deg_ref[1, :, 0:1] + 1.0
    dinv = lax.rsqrt(deg)
    xl = lax.dot_general(x_ref[...], w_ref[...], (((1,), (1,)), ((), ())),
                         preferred_element_type=jnp.float32)
    y_ref[...] = dinv * xl


BRB = 2528

_y_call = pl.pallas_call(
    _y_body,
    grid=(N_PAD // BRB,),
    in_specs=[
        pl.BlockSpec((BRB, D), lambda i: (i, 0)),
        pl.BlockSpec((D, D), lambda i: (0, 0)),
        pl.BlockSpec((NC, BRB, 16), lambda i: (0, i, 0)),
    ],
    out_specs=pl.BlockSpec((BRB, D), lambda i: (i, 0)),
    out_shape=jax.ShapeDtypeStruct((N_PAD, D), jnp.float32),
)


BR = 1000


def _agg_body(s_ref, y_ref, deg_ref, b_ref, agg_ref, stats_ref):
    i = pl.program_id(0)
    deg = deg_ref[0, :, 0:1] + deg_ref[1, :, 0:1] + 1.0
    dinv = lax.rsqrt(deg)
    agg = dinv * (s_ref[0] + s_ref[1] + y_ref[...]) + b_ref[...]
    agg_ref[...] = agg
    st = jnp.concatenate(
        [jnp.sum(agg, axis=0, keepdims=True),
         jnp.sum(agg * agg, axis=0, keepdims=True)], axis=0)

    @pl.when(i == 0)
    def _init():
        stats_ref[...] = st

    @pl.when(i > 0)
    def _acc():
        stats_ref[...] += st


_agg_call = pl.pallas_call(
    _agg_body,
    grid=(N // BR,),
    in_specs=[
        pl.BlockSpec((NC, BR, D), lambda i: (0, i, 0)),
        pl.BlockSpec((BR, D), lambda i: (i, 0)),
        pl.BlockSpec((NC, BR, 16), lambda i: (0, i, 0)),
        pl.BlockSpec((1, D), lambda i: (0, 0)),
    ],
    out_specs=[
        pl.BlockSpec((BR, D), lambda i: (i, 0)),
        pl.BlockSpec((2, D), lambda i: (0, 0)),
    ],
    out_shape=[
        jax.ShapeDtypeStruct((N, D), jnp.float32),
        jax.ShapeDtypeStruct((2, D), jnp.float32),
    ],
)


def _fin_body(agg_ref, stats_ref, g_ref, bt_ref, x_ref, o_ref):
    mean = stats_ref[0:1, :] * (1.0 / N)
    var = stats_ref[1:2, :] * (1.0 / N) - mean * mean
    inv = lax.rsqrt(var + 1e-5)
    h = (agg_ref[...] - mean) * inv * g_ref[...] + bt_ref[...]
    h = jnp.where(h >= 0, h, 0.1 * h)
    h = h + x_ref[...]
    o_ref[...] = jnp.where(h >= 0, h, 0.1 * h)


_fin_call = pl.pallas_call(
    _fin_body,
    grid=(N // BR,),
    in_specs=[
        pl.BlockSpec((BR, D), lambda i: (i, 0)),
        pl.BlockSpec((2, D), lambda i: (0, 0)),
        pl.BlockSpec((1, D), lambda i: (0, 0)),
        pl.BlockSpec((1, D), lambda i: (0, 0)),
        pl.BlockSpec((BR, D), lambda i: (i, 0)),
    ],
    out_specs=pl.BlockSpec((BR, D), lambda i: (i, 0)),
    out_shape=jax.ShapeDtypeStruct((N, D), jnp.float32),
)


def kernel(x, W, b, bn_gamma, bn_beta, edge_index):
    ei = edge_index.astype(jnp.int32)
    pad = jnp.full((E_PAD - E,), N, jnp.int32)
    rows = jnp.concatenate([ei[0], pad])
    cols = jnp.concatenate([ei[1], pad])
    rows2d = rows.reshape(E_PAD // CHUNK, CHUNK)
    ones16 = jnp.ones((CHUNK, 16), jnp.float32)
    z16 = jnp.zeros((N_PAD, 16), jnp.float32)
    x_pad = jnp.concatenate([x, jnp.zeros((N_PAD - N, D), x.dtype)])

    deg_kernel, seg_kernel = _sc_kernels()
    deg = deg_kernel(rows2d, ones16, z16)
    y = _y_call(x_pad, W, deg)
    s = seg_kernel(rows, cols, y)
    agg, stats = _agg_call(s, y, deg, b.reshape(1, D))
    return _fin_call(agg, stats, bn_gamma.reshape(1, D),
                     bn_beta.reshape(1, D), x)


# speedup vs baseline: 2.1594x; 1.0528x over previous
import functools

import jax
import jax.numpy as jnp
from jax import lax
from jax.experimental import pallas as pl
from jax.experimental.pallas import tpu as pltpu
from jax.experimental.pallas import tpu_sc as plsc

N = 10000
E = 320000
D = 128
NC = 2
NS = 16
NW = NC * NS
CHUNK = 128
CHUNKS_PER_W = 79
EPW = CHUNKS_PER_W * CHUNK
E_PAD = EPW * NW
CH0 = 118
CH1 = 2 * CHUNKS_PER_W - CH0
N_PAD = N + 112
RPT = N_PAD // NS


def _deg_body(rows_hbm, ones_hbm, z16_hbm, deg_out, ridx_all, ones_v, deg_sh):
    c = lax.axis_index("c")
    s = lax.axis_index("s")
    w = c * NS + s
    pltpu.sync_copy(ones_hbm, ones_v)
    pltpu.sync_copy(rows_hbm.at[pl.ds(w * CHUNKS_PER_W, CHUNKS_PER_W)],
                    ridx_all)
    pltpu.sync_copy(z16_hbm.at[pl.ds(s * RPT, RPT)],
                    deg_sh.at[pl.ds(s * RPT, RPT)])
    plsc.subcore_barrier()

    def body(i, carry):
        pltpu.sync_copy(ones_v, deg_sh.at[ridx_all.at[i]], add=True)
        return carry

    lax.fori_loop(0, CHUNKS_PER_W, body, 0)
    plsc.subcore_barrier()
    pltpu.sync_copy(deg_sh.at[pl.ds(s * RPT, RPT)],
                    deg_out.at[c, pl.ds(s * RPT, RPT)])


def _seg_body(rows_hbm, cols_hbm, y_hbm, s_out,
              ridx_v, cidx_v, ridx_v1, cidx_v1, gbuf, gbuf1, s_sh, sem, sem1):
    c = lax.axis_index("c")
    s = lax.axis_index("s")
    w = c * NS + s

    zero = jnp.zeros((16,), jnp.float32)

    def zrow(i, carry):
        for j in range(D // 16):
            gbuf[i, pl.ds(j * 16, 16)] = zero
        return carry

    lax.fori_loop(0, CHUNK, zrow, 0)
    for off in range(0, RPT - CHUNK + 1, CHUNK):
        pltpu.sync_copy(gbuf, s_sh.at[pl.ds(s * RPT + off, CHUNK)])
    _tail = RPT % CHUNK
    if _tail:
        pltpu.sync_copy(gbuf.at[pl.ds(0, _tail)],
                        s_sh.at[pl.ds(s * RPT + RPT - _tail, _tail)])
    plsc.subcore_barrier()

    chunk0 = jnp.where(c == 0, s * CH0, NS * CH0 + s * CH1)
    npairs = jnp.where(c == 0, CH0 // 2, CH1 // 2)

    def load_idx(i, ridx, cidx):
        base = (chunk0 + i) * CHUNK
        pltpu.sync_copy(rows_hbm.at[pl.ds(base, CHUNK)], ridx)
        pltpu.sync_copy(cols_hbm.at[pl.ds(base, CHUNK)], cidx)

    load_idx(0, ridx_v, cidx_v)
    pltpu.async_copy(y_hbm.at[ridx_v], gbuf, sem)

    def body(p, carry):
        i0 = 2 * p
        load_idx(i0 + 1, ridx_v1, cidx_v1)
        pltpu.async_copy(y_hbm.at[ridx_v1], gbuf1, sem1)
        pltpu.make_async_copy(y_hbm.at[ridx_v], gbuf, sem).wait()
        pltpu.sync_copy(gbuf, s_sh.at[cidx_v], add=True)

        @pl.when(p < npairs - 1)
        def _():
            load_idx(i0 + 2, ridx_v, cidx_v)
            pltpu.async_copy(y_hbm.at[ridx_v], gbuf, sem)

        pltpu.make_async_copy(y_hbm.at[ridx_v1], gbuf1, sem1).wait()
        pltpu.sync_copy(gbuf1, s_sh.at[cidx_v1], add=True)
        return carry

    lax.fori_loop(0, npairs, body, 0)
    plsc.subcore_barrier()
    pltpu.sync_copy(s_sh.at[pl.ds(s * RPT, RPT)],
                    s_out.at[c, pl.ds(s * RPT, RPT)])


@functools.cache
def _sc_kernels():
    mesh = plsc.VectorSubcoreMesh(
        core_axis_name="c", subcore_axis_name="s",
        num_cores=NC, num_subcores=NS)
    deg_kernel = pl.kernel(
        _deg_body,
        out_type=jax.ShapeDtypeStruct((NC, N_PAD, 16), jnp.float32),
        mesh=mesh,
        compiler_params=pltpu.CompilerParams(use_tc_tiling_on_sc=False),
        scratch_types=[
            pltpu.VMEM((CHUNKS_PER_W, CHUNK), jnp.int32),
            pltpu.VMEM((CHUNK, 16), jnp.float32),
            pltpu.VMEM_SHARED((N_PAD, 16), jnp.float32),
        ],
    )
    seg_kernel = pl.kernel(
        _seg_body,
        out_type=jax.ShapeDtypeStruct((NC, N_PAD, D), jnp.float32),
        mesh=mesh,
        scratch_types=[
            pltpu.VMEM((CHUNK,), jnp.int32),
            pltpu.VMEM((CHUNK,), jnp.int32),
            pltpu.VMEM((CHUNK,), jnp.int32),
            pltpu.VMEM((CHUNK,), jnp.int32),
            pltpu.VMEM((CHUNK, D), jnp.float32),
            pltpu.VMEM((CHUNK, D), jnp.float32),
            pltpu.VMEM_SHARED((N_PAD, D), jnp.float32),
            pltpu.SemaphoreType.DMA,
            pltpu.SemaphoreType.DMA,
        ],
    )
    return deg_kernel, seg_kernel


def _y_body(x_ref, w_ref, deg_ref, y_ref):
    deg = deg_ref[0, :, 0:1] + deg_ref[1, :, 0:1] + 1.0
    dinv = lax.rsqrt(deg)
    xl = lax.dot_general(x_ref[...], w_ref[...], (((1,), (1,)), ((), ())),
                         preferred_element_type=jnp.float32)
    y_ref[...] = dinv * xl


BRB = 2528

_y_call = pl.pallas_call(
    _y_body,
    grid=(N_PAD // BRB,),
    in_specs=[
        pl.BlockSpec((BRB, D), lambda i: (i, 0)),
        pl.BlockSpec((D, D), lambda i: (0, 0)),
        pl.BlockSpec((NC, BRB, 16), lambda i: (0, i, 0)),
    ],
    out_specs=pl.BlockSpec((BRB, D), lambda i: (i, 0)),
    out_shape=jax.ShapeDtypeStruct((N_PAD, D), jnp.float32),
)


BR = 1000


def _agg_body(s_ref, y_ref, deg_ref, b_ref, agg_ref, stats_ref):
    i = pl.program_id(0)
    deg = deg_ref[0, :, 0:1] + deg_ref[1, :, 0:1] + 1.0
    dinv = lax.rsqrt(deg)
    agg = dinv * (s_ref[0] + s_ref[1] + y_ref[...]) + b_ref[...]
    agg_ref[...] = agg
    st = jnp.concatenate(
        [jnp.sum(agg, axis=0, keepdims=True),
         jnp.sum(agg * agg, axis=0, keepdims=True)], axis=0)

    @pl.when(i == 0)
    def _init():
        stats_ref[...] = st

    @pl.when(i > 0)
    def _acc():
        stats_ref[...] += st


_agg_call = pl.pallas_call(
    _agg_body,
    grid=(N // BR,),
    in_specs=[
        pl.BlockSpec((NC, BR, D), lambda i: (0, i, 0)),
        pl.BlockSpec((BR, D), lambda i: (i, 0)),
        pl.BlockSpec((NC, BR, 16), lambda i: (0, i, 0)),
        pl.BlockSpec((1, D), lambda i: (0, 0)),
    ],
    out_specs=[
        pl.BlockSpec((BR, D), lambda i: (i, 0)),
        pl.BlockSpec((2, D), lambda i: (0, 0)),
    ],
    out_shape=[
        jax.ShapeDtypeStruct((N, D), jnp.float32),
        jax.ShapeDtypeStruct((2, D), jnp.float32),
    ],
)


def _fin_body(agg_ref, stats_ref, g_ref, bt_ref, x_ref, o_ref):
    mean = stats_ref[0:1, :] * (1.0 / N)
    var = stats_ref[1:2, :] * (1.0 / N) - mean * mean
    inv = lax.rsqrt(var + 1e-5)
    h = (agg_ref[...] - mean) * inv * g_ref[...] + bt_ref[...]
    h = jnp.where(h >= 0, h, 0.1 * h)
    h = h + x_ref[...]
    o_ref[...] = jnp.where(h >= 0, h, 0.1 * h)


_fin_call = pl.pallas_call(
    _fin_body,
    grid=(N // BR,),
    in_specs=[
        pl.BlockSpec((BR, D), lambda i: (i, 0)),
        pl.BlockSpec((2, D), lambda i: (0, 0)),
        pl.BlockSpec((1, D), lambda i: (0, 0)),
        pl.BlockSpec((1, D), lambda i: (0, 0)),
        pl.BlockSpec((BR, D), lambda i: (i, 0)),
    ],
    out_specs=pl.BlockSpec((BR, D), lambda i: (i, 0)),
    out_shape=jax.ShapeDtypeStruct((N, D), jnp.float32),
)


def kernel(x, W, b, bn_gamma, bn_beta, edge_index):
    ei = edge_index.astype(jnp.int32)
    pad = jnp.full((E_PAD - E,), N, jnp.int32)
    rows = jnp.concatenate([ei[0], pad])
    cols = jnp.concatenate([ei[1], pad])
    rows2d = rows.reshape(E_PAD // CHUNK, CHUNK)
    ones16 = jnp.ones((CHUNK, 16), jnp.float32)
    z16 = jnp.zeros((N_PAD, 16), jnp.float32)
    x_pad = jnp.concatenate([x, jnp.zeros((N_PAD - N, D), x.dtype)])

    deg_kernel, seg_kernel = _sc_kernels()
    deg = deg_kernel(rows2d, ones16, z16)
    y = _y_call(x_pad, W, deg)
    s = seg_kernel(rows, cols, y)
    agg, stats = _agg_call(s, y, deg, b.reshape(1, D))
    return _fin_call(agg, stats, bn_gamma.reshape(1, D),
                     bn_beta.reshape(1, D), x)
